# Initial kernel scaffold; baseline (speedup 1.0000x reference)
import jax
import jax.numpy as jnp
from jax.experimental import pallas as pl


def kernel(features, edge_index, neg_edge_index, W1, b1, W2, b2):
    raise NotImplementedError("write your pallas kernel here")


# speedup vs baseline: 3.7798x; 3.7798x over previous
import functools

import jax
import jax.numpy as jnp
from jax import lax
from jax.experimental import pallas as pl
from jax.experimental.pallas import tpu as pltpu
from jax.experimental.pallas import tpu_sc as plsc

NN = 10000
DD = 256
TAU = 0.1
NPAD = 10240
NC, NS = 2, 16
RPT = NPAD // NS
GB = 128


def _sc_mesh():
    return plsc.VectorSubcoreMesh(core_axis_name="c", subcore_axis_name="s")


def _hist(dstp, zrows, orows):
    g = dstp.shape[1]

    @functools.partial(
        pl.kernel,
        out_type=jax.ShapeDtypeStruct((NPAD, 128), jnp.float32),
        mesh=_sc_mesh(),
        scratch_types=[
            pltpu.VMEM_SHARED((NPAD, 128), jnp.float32),
            pltpu.VMEM((g, 128), jnp.int32),
            pltpu.VMEM((GB, 128), jnp.float32),
        ],
    )
    def k(dst_hbm, z_hbm, o_hbm, out_hbm, acc, idst, ones_v):
        c = lax.axis_index("c")
        s = lax.axis_index("s")
        pltpu.sync_copy(z_hbm, acc.at[pl.ds(s * RPT, RPT)])
        pltpu.sync_copy(o_hbm, ones_v)
        pltpu.sync_copy(dst_hbm.at[s], idst)
        plsc.subcore_barrier()

        def body(j, carry):
            pltpu.sync_copy(ones_v, acc.at[idst.at[j]], add=True)
            return carry

        lax.fori_loop(0, g, body, 0)
        plsc.subcore_barrier()

        @pl.when(c == 0)
        def _():
            pltpu.sync_copy(acc.at[pl.ds(s * RPT, RPT)],
                            out_hbm.at[pl.ds(s * RPT, RPT)])

    return k(dstp, zrows, orows)


def _segsum(xwh, srcp, dstp):
    g = srcp.shape[1]

    @functools.partial(
        pl.kernel,
        out_type=jax.ShapeDtypeStruct((NC, NPAD, 128), jnp.float32),
        mesh=_sc_mesh(),
        scratch_types=[
            pltpu.VMEM_SHARED((NPAD, 128), jnp.float32),
            pltpu.VMEM((g, 128), jnp.int32),
            pltpu.VMEM((g, 128), jnp.int32),
            pltpu.VMEM((GB, 128), jnp.float32),
            pltpu.SemaphoreType.DMA,
        ],
    )
    def k(xwh_hbm, src_hbm, dst_hbm, out_hbm, acc, isrc, idst, rows, sem):
        c = lax.axis_index("c")
        s = lax.axis_index("s")
        pltpu.sync_copy(xwh_hbm.at[c, pl.ds(s * RPT, RPT)],
                        acc.at[pl.ds(s * RPT, RPT)])
        pltpu.sync_copy(src_hbm.at[s], isrc)
        pltpu.sync_copy(dst_hbm.at[s], idst)
        plsc.subcore_barrier()

        def body(j, carry):
            pltpu.async_copy(xwh_hbm.at[c].at[isrc.at[j]], rows, sem).wait()
            pltpu.sync_copy(rows, acc.at[idst.at[j]], add=True)
            return carry

        lax.fori_loop(0, g, body, 0)
        plsc.subcore_barrier()
        pltpu.sync_copy(acc.at[pl.ds(s * RPT, RPT)],
                        out_hbm.at[c, pl.ds(s * RPT, RPT)])

    return k(xwh, srcp, dstp)


def _edge_dots(reps, ea, eb):
    g3 = ea.shape[1]
    eout = NC * NS * g3 * GB

    @functools.partial(
        pl.kernel,
        out_type=jax.ShapeDtypeStruct((eout, 16), jnp.float32),
        mesh=_sc_mesh(),
        scratch_types=[
            pltpu.VMEM((g3, 128), jnp.int32),
            pltpu.VMEM((g3, 128), jnp.int32),
            pltpu.VMEM((GB, 256), jnp.float32),
            pltpu.VMEM((GB, 256), jnp.float32),
            pltpu.VMEM((GB, 16), jnp.float32),
            pltpu.SemaphoreType.DMA,
            pltpu.SemaphoreType.DMA,
        ],
    )
    def k(reps_hbm, ea_hbm, eb_hbm, out_hbm, ia, ib, ra, rb, dv, sema, semb):
        c = lax.axis_index("c")
        s = lax.axis_index("s")
        w = c * NS + s
        pltpu.sync_copy(ea_hbm.at[w], ia)
        pltpu.sync_copy(eb_hbm.at[w], ib)

        def body(j, carry):
            da = pltpu.async_copy(reps_hbm.at[ia.at[j]], ra, sema)
            db = pltpu.async_copy(reps_hbm.at[ib.at[j]], rb, semb)
            da.wait()
            db.wait()

            def edge(e, ecarry):
                acc = ra[e, pl.ds(0, 16)] * rb[e, pl.ds(0, 16)]
                for v in range(1, 16):
                    acc = acc + ra[e, pl.ds(v * 16, 16)] * rb[e, pl.ds(v * 16, 16)]
                dv[e, :] = acc
                return ecarry

            lax.fori_loop(0, GB, edge, 0)
            pltpu.sync_copy(dv, out_hbm.at[pl.ds((w * g3 + j) * GB, GB)])
            return carry

        lax.fori_loop(0, g3, body, 0)

    return k(reps, ea, eb)


def _dinv_block(hist_ref, i):
    deg = hist_ref[:, 0:1] + 1.0
    row = lax.broadcasted_iota(jnp.int32, (256, 1), 0) + i * 256
    return jnp.where(row < NN, lax.rsqrt(deg), 0.0)


def _mm_scale(x, w, b2d, hist, relu_in):
    nblk = NPAD // 256

    def body(x_ref, w_ref, b_ref, h_ref, out_ref):
        i = pl.program_id(0)
        dv = _dinv_block(h_ref, i)
        if relu_in:
            xin = jnp.concatenate([x_ref[0], x_ref[1]], axis=1)
            xin = jnp.maximum(xin * dv, 0.0)
        else:
            xin = x_ref[...]
        xw = jnp.dot(xin, w_ref[...], preferred_element_type=jnp.float32)
        s = (xw + b_ref[...]) * dv
        out_ref[0] = s[:, :128]
        out_ref[1] = s[:, 128:]

    in_spec0 = (pl.BlockSpec((NC, 256, 128), lambda i: (0, i, 0)) if relu_in
                else pl.BlockSpec((256, 256), lambda i: (i, 0)))
    return pl.pallas_call(
        body,
        grid=(nblk,),
        in_specs=[
            in_spec0,
            pl.BlockSpec((256, 256), lambda i: (0, 0)),
            pl.BlockSpec((1, 256), lambda i: (0, 0)),
            pl.BlockSpec((256, 128), lambda i: (i, 0)),
        ],
        out_specs=pl.BlockSpec((NC, 256, 128), lambda i: (0, i, 0)),
        out_shape=jax.ShapeDtypeStruct((NC, NPAD, 128), jnp.float32),
    )(x, w, b2d, hist)


def _normalize(presum, hist):
    nblk = NPAD // 256

    def body(p_ref, h_ref, out_ref):
        i = pl.program_id(0)
        dv = _dinv_block(h_ref, i)
        r = jnp.concatenate([p_ref[0], p_ref[1]], axis=1) * dv
        nrm = jnp.sqrt(jnp.sum(r * r, axis=1, keepdims=True))
        out_ref[...] = r / jnp.maximum(nrm, 1e-12)

    return pl.pallas_call(
        body,
        grid=(nblk,),
        in_specs=[
            pl.BlockSpec((NC, 256, 128), lambda i: (0, i, 0)),
            pl.BlockSpec((256, 128), lambda i: (i, 0)),
        ],
        out_specs=pl.BlockSpec((256, 256), lambda i: (i, 0)),
        out_shape=jax.ShapeDtypeStruct((NPAD, 256), jnp.float32),
    )(presum, hist)


def _loss(posd, negd, e0r, e1r, n0r, n1r):
    nblk, blk = posd.shape[0], posd.shape[1]

    def body(pd_ref, nd_ref, e0_ref, e1_ref, n0_ref, n1_ref, est_ref, acc_ref):
        g = pl.program_id(0)
        pd = jnp.sum(pd_ref[0], axis=1)
        nd = jnp.sum(nd_ref[0], axis=1)
        pm = (e0_ref[0, 0] < e1_ref[0, 0]).astype(jnp.float32)
        nm = (n0_ref[0, 0] < n1_ref[0, 0]).astype(jnp.float32)
        est = jnp.maximum(pd, 0.0)
        est_ref[0, 0] = jnp.where(est < TAU, 0.0, est)

        @pl.when(g == 0)
        def _():
            acc_ref[0] = 0.0
            acc_ref[1] = 0.0
            acc_ref[2] = 0.0
            acc_ref[3] = 0.0

        acc_ref[0] += jnp.sum(pm * (pd - 1.0) ** 2)
        acc_ref[1] += jnp.sum(nm * nd * nd)
        acc_ref[2] += jnp.sum(pm)
        acc_ref[3] += jnp.sum(nm)

    idx_spec = pl.BlockSpec((1, 1, blk), lambda g: (g, 0, 0))
    return pl.pallas_call(
        body,
        grid=(nblk,),
        in_specs=[
            pl.BlockSpec((1, blk, 16), lambda g: (g, 0, 0)),
            pl.BlockSpec((1, blk, 16), lambda g: (g, 0, 0)),
            idx_spec, idx_spec, idx_spec, idx_spec,
        ],
        out_specs=[
            pl.BlockSpec((1, 1, blk), lambda g: (g, 0, 0)),
            pl.BlockSpec(memory_space=pltpu.SMEM),
        ],
        out_shape=[
            jax.ShapeDtypeStruct((nblk, 1, blk), jnp.float32),
            jax.ShapeDtypeStruct((4,), jnp.float32),
        ],
    )(posd, negd, e0r, e1r, n0r, n1r)


def _pad_edges(idx, ntiles, group):
    k = idx.shape[0]
    per = -(-k // (ntiles * group))
    tot = ntiles * per * group
    pad = jnp.full((tot - k,), NPAD - 1, jnp.int32)
    return jnp.concatenate([idx, pad]).reshape(ntiles, per, group)


def kernel(features, edge_index, neg_edge_index, W1, b1, W2, b2):
    e = edge_index.shape[1]
    src = edge_index[0].astype(jnp.int32)
    dst = edge_index[1].astype(jnp.int32)

    srcp = _pad_edges(src, NS, GB)
    dstp = _pad_edges(dst, NS, GB)
    zrows = jnp.zeros((RPT, 128), jnp.float32)
    orows = jnp.ones((GB, 128), jnp.float32)
    hist = _hist(dstp, zrows, orows)

    fpad = jnp.zeros((NPAD, DD), jnp.float32).at[:NN].set(features)
    xw1h = _mm_scale(fpad, W1, b1.reshape(1, DD), hist, relu_in=False)

    ps1 = _segsum(xw1h, srcp, dstp)
    xw2h = _mm_scale(ps1, W2, b2.reshape(1, DD), hist, relu_in=True)
    ps2 = _segsum(xw2h, srcp, dstp)

    repspad = _normalize(ps2, hist)

    ea = _pad_edges(jnp.concatenate([src, neg_edge_index[0].astype(jnp.int32)]),
                    NC * NS, GB)
    eb = _pad_edges(jnp.concatenate([dst, neg_edge_index[1].astype(jnp.int32)]),
                    NC * NS, GB)
    dots = _edge_dots(repspad, ea, eb)

    nblk = 80
    blk = e // nblk
    posd = dots[:e].reshape(nblk, blk, 16)
    negd = dots[e:2 * e].reshape(nblk, blk, 16)
    e0r = src.reshape(nblk, 1, blk)
    e1r = dst.reshape(nblk, 1, blk)
    n0r = neg_edge_index[0].astype(jnp.int32).reshape(nblk, 1, blk)
    n1r = neg_edge_index[1].astype(jnp.int32).reshape(nblk, 1, blk)
    est3, accs = _loss(posd, negd, e0r, e1r, n0r, n1r)

    reps = repspad[:NN]
    est = est3.reshape(e)
    rec_loss = (accs[1] + accs[0]) * NN / (accs[2] + accs[3])
    return reps, rec_loss, est


# speedup vs baseline: 3.9592x; 1.0475x over previous
import functools

import jax
import jax.numpy as jnp
from jax import lax
from jax.experimental import pallas as pl
from jax.experimental.pallas import tpu as pltpu
from jax.experimental.pallas import tpu_sc as plsc

NN = 10000
DD = 256
TAU = 0.1
NPAD = 10240
NC, NS = 2, 16
RPT = NPAD // NS
GB = 128


def _sc_mesh():
    return plsc.VectorSubcoreMesh(core_axis_name="c", subcore_axis_name="s")


def _hist(dstp, zrows, orows):
    g = dstp.shape[1]

    @functools.partial(
        pl.kernel,
        out_type=jax.ShapeDtypeStruct((NC, NPAD, 128), jnp.float32),
        mesh=_sc_mesh(),
        scratch_types=[
            pltpu.VMEM_SHARED((NPAD, 128), jnp.float32),
            pltpu.VMEM((g, 128), jnp.int32),
            pltpu.VMEM((GB, 128), jnp.float32),
        ],
    )
    def k(dst_hbm, z_hbm, o_hbm, out_hbm, acc, idst, ones_v):
        c = lax.axis_index("c")
        s = lax.axis_index("s")
        w = c * NS + s
        pltpu.sync_copy(z_hbm, acc.at[pl.ds(s * RPT, RPT)])
        pltpu.sync_copy(o_hbm, ones_v)
        pltpu.sync_copy(dst_hbm.at[w], idst)
        plsc.subcore_barrier()

        def body(j, carry):
            pltpu.sync_copy(ones_v, acc.at[idst.at[j]], add=True)
            return carry

        lax.fori_loop(0, g, body, 0)
        plsc.subcore_barrier()
        pltpu.sync_copy(acc.at[pl.ds(s * RPT, RPT)],
                        out_hbm.at[c, pl.ds(s * RPT, RPT)])

    return k(dstp, zrows, orows)


def _segsum(xwh, srcp, dstp):
    g, gb = srcp.shape[1], srcp.shape[2]
    ch_g = 16
    assert g % ch_g == 0 and ch_g % 8 == 0

    @functools.partial(
        pl.kernel,
        out_type=jax.ShapeDtypeStruct((NC, NPAD, 128), jnp.float32),
        mesh=_sc_mesh(),
        scratch_types=[
            pltpu.VMEM_SHARED((NPAD, 128), jnp.float32),
            pltpu.VMEM((ch_g, gb), jnp.int32),
            pltpu.VMEM((ch_g, gb), jnp.int32),
            pltpu.VMEM((gb, 128), jnp.float32),
            pltpu.VMEM((gb, 128), jnp.float32),
            pltpu.SemaphoreType.DMA,
            pltpu.SemaphoreType.DMA,
        ],
    )
    def k(xwh_hbm, src_hbm, dst_hbm, out_hbm, acc, isrc, idst, rowsa, rowsb,
          sema, semb):
        c = lax.axis_index("c")
        s = lax.axis_index("s")
        pltpu.sync_copy(xwh_hbm.at[c, pl.ds(s * RPT, RPT)],
                        acc.at[pl.ds(s * RPT, RPT)])
        plsc.subcore_barrier()
        xwc = xwh_hbm.at[c]

        def chunk(ci, carry):
            pltpu.sync_copy(src_hbm.at[s, pl.ds(ci * ch_g, ch_g)], isrc)
            pltpu.sync_copy(dst_hbm.at[s, pl.ds(ci * ch_g, ch_g)], idst)
            pltpu.async_copy(xwc.at[isrc.at[0]], rowsa, sema)
            pltpu.async_copy(xwc.at[isrc.at[1]], rowsb, semb)

            def body(i, c2):
                j = 2 * i
                pltpu.make_async_copy(xwc.at[isrc.at[0]], rowsa, sema).wait()
                pltpu.sync_copy(rowsa, acc.at[idst.at[j]], add=True)

                @pl.when(j + 2 < ch_g)
                def _():
                    pltpu.async_copy(xwc.at[isrc.at[j + 2]], rowsa, sema)

                pltpu.make_async_copy(xwc.at[isrc.at[0]], rowsb, semb).wait()
                pltpu.sync_copy(rowsb, acc.at[idst.at[j + 1]], add=True)

                @pl.when(j + 3 < ch_g)
                def _():
                    pltpu.async_copy(xwc.at[isrc.at[j + 3]], rowsb, semb)

                return c2

            lax.fori_loop(0, ch_g // 2, body, 0)
            return carry

        lax.fori_loop(0, g // ch_g, chunk, 0)
        plsc.subcore_barrier()
        pltpu.sync_copy(acc.at[pl.ds(s * RPT, RPT)],
                        out_hbm.at[c, pl.ds(s * RPT, RPT)])

    return k(xwh, srcp, dstp)


def _edge_dots(reps, ea, eb):
    g3, gb = ea.shape[1], ea.shape[2]
    eout = NC * NS * g3 * gb

    @functools.partial(
        pl.kernel,
        out_type=jax.ShapeDtypeStruct((eout, 16), jnp.float32),
        mesh=_sc_mesh(),
        scratch_types=[
            pltpu.VMEM((g3, gb), jnp.int32),
            pltpu.VMEM((g3, gb), jnp.int32),
            pltpu.VMEM((gb, 256), jnp.float32),
            pltpu.VMEM((gb, 256), jnp.float32),
            pltpu.VMEM((gb, 256), jnp.float32),
            pltpu.VMEM((gb, 256), jnp.float32),
            pltpu.VMEM((gb, 16), jnp.float32),
            pltpu.VMEM((gb, 16), jnp.float32),
            pltpu.SemaphoreType.DMA,
            pltpu.SemaphoreType.DMA,
        ],
    )
    def k(reps_hbm, ea_hbm, eb_hbm, out_hbm, ia, ib, raa, rba, rab, rbb,
          dva, dvb, sema, semb):
        c = lax.axis_index("c")
        s = lax.axis_index("s")
        w = c * NS + s
        pltpu.sync_copy(ea_hbm.at[w], ia)
        pltpu.sync_copy(eb_hbm.at[w], ib)
        pltpu.async_copy(reps_hbm.at[ia.at[0]], raa, sema)
        pltpu.async_copy(reps_hbm.at[ib.at[0]], rba, sema)
        pltpu.async_copy(reps_hbm.at[ia.at[1]], rab, semb)
        pltpu.async_copy(reps_hbm.at[ib.at[1]], rbb, semb)

        def dot_group(ra, rb, dv):
            def edge(e, ecarry):
                acc = ra[e, pl.ds(0, 16)] * rb[e, pl.ds(0, 16)]
                for v in range(1, 16):
                    acc = acc + ra[e, pl.ds(v * 16, 16)] * rb[e, pl.ds(v * 16, 16)]
                dv[e, :] = acc
                return ecarry

            lax.fori_loop(0, gb, edge, 0)

        def body(i, carry):
            j = 2 * i
            pltpu.make_async_copy(reps_hbm.at[ia.at[0]], raa, sema).wait()
            pltpu.make_async_copy(reps_hbm.at[ia.at[0]], rba, sema).wait()
            dot_group(raa, rba, dva)

            @pl.when(j + 2 < g3)
            def _():
                pltpu.async_copy(reps_hbm.at[ia.at[j + 2]], raa, sema)
                pltpu.async_copy(reps_hbm.at[ib.at[j + 2]], rba, sema)

            pltpu.sync_copy(dva, out_hbm.at[pl.ds((w * g3 + j) * gb, gb)])

            pltpu.make_async_copy(reps_hbm.at[ia.at[0]], rab, semb).wait()
            pltpu.make_async_copy(reps_hbm.at[ia.at[0]], rbb, semb).wait()
            dot_group(rab, rbb, dvb)

            @pl.when(j + 3 < g3)
            def _():
                pltpu.async_copy(reps_hbm.at[ia.at[j + 3]], rab, semb)
                pltpu.async_copy(reps_hbm.at[ib.at[j + 3]], rbb, semb)

            pltpu.sync_copy(dvb, out_hbm.at[pl.ds((w * g3 + j + 1) * gb, gb)])
            return carry

        lax.fori_loop(0, g3 // 2, body, 0)

    return k(reps, ea, eb)


def _dinv_block(hist_ref, i):
    deg = hist_ref[0, :, 0:1] + hist_ref[1, :, 0:1] + 1.0
    row = lax.broadcasted_iota(jnp.int32, (256, 1), 0) + i * 256
    return jnp.where(row < NN, lax.rsqrt(deg), 0.0)


def _mm_scale(x, w, b2d, hist, relu_in):
    nblk = NPAD // 256

    def body(x_ref, w_ref, b_ref, h_ref, out_ref):
        i = pl.program_id(0)
        dv = _dinv_block(h_ref, i)
        if relu_in:
            xin = jnp.concatenate([x_ref[0], x_ref[1]], axis=1)
            xin = jnp.maximum(xin * dv, 0.0)
        else:
            xin = x_ref[...]
        xw = jnp.dot(xin, w_ref[...], preferred_element_type=jnp.float32)
        s = (xw + b_ref[...]) * dv
        out_ref[0] = s[:, :128]
        out_ref[1] = s[:, 128:]

    in_spec0 = (pl.BlockSpec((NC, 256, 128), lambda i: (0, i, 0)) if relu_in
                else pl.BlockSpec((256, 256), lambda i: (i, 0)))
    return pl.pallas_call(
        body,
        grid=(nblk,),
        in_specs=[
            in_spec0,
            pl.BlockSpec((256, 256), lambda i: (0, 0)),
            pl.BlockSpec((1, 256), lambda i: (0, 0)),
            pl.BlockSpec((NC, 256, 128), lambda i: (0, i, 0)),
        ],
        out_specs=pl.BlockSpec((NC, 256, 128), lambda i: (0, i, 0)),
        out_shape=jax.ShapeDtypeStruct((NC, NPAD, 128), jnp.float32),
    )(x, w, b2d, hist)


def _normalize(presum, hist):
    nblk = NPAD // 256

    def body(p_ref, h_ref, out_ref):
        i = pl.program_id(0)
        dv = _dinv_block(h_ref, i)
        r = jnp.concatenate([p_ref[0], p_ref[1]], axis=1) * dv
        nrm = jnp.sqrt(jnp.sum(r * r, axis=1, keepdims=True))
        out_ref[...] = r / jnp.maximum(nrm, 1e-12)

    return pl.pallas_call(
        body,
        grid=(nblk,),
        in_specs=[
            pl.BlockSpec((NC, 256, 128), lambda i: (0, i, 0)),
            pl.BlockSpec((NC, 256, 128), lambda i: (0, i, 0)),
        ],
        out_specs=pl.BlockSpec((256, 256), lambda i: (i, 0)),
        out_shape=jax.ShapeDtypeStruct((NPAD, 256), jnp.float32),
    )(presum, hist)


def _loss(posd, negd, e0r, e1r, n0r, n1r):
    nblk, blk = posd.shape[0], posd.shape[1]

    def body(pd_ref, nd_ref, e0_ref, e1_ref, n0_ref, n1_ref, est_ref, acc_ref):
        g = pl.program_id(0)
        pd = jnp.sum(pd_ref[0], axis=1)
        nd = jnp.sum(nd_ref[0], axis=1)
        pm = (e0_ref[0, 0] < e1_ref[0, 0]).astype(jnp.float32)
        nm = (n0_ref[0, 0] < n1_ref[0, 0]).astype(jnp.float32)
        est = jnp.maximum(pd, 0.0)
        est_ref[0, 0] = jnp.where(est < TAU, 0.0, est)

        @pl.when(g == 0)
        def _():
            acc_ref[0] = 0.0
            acc_ref[1] = 0.0
            acc_ref[2] = 0.0
            acc_ref[3] = 0.0

        acc_ref[0] += jnp.sum(pm * (pd - 1.0) ** 2)
        acc_ref[1] += jnp.sum(nm * nd * nd)
        acc_ref[2] += jnp.sum(pm)
        acc_ref[3] += jnp.sum(nm)

    idx_spec = pl.BlockSpec((1, 1, blk), lambda g: (g, 0, 0))
    return pl.pallas_call(
        body,
        grid=(nblk,),
        in_specs=[
            pl.BlockSpec((1, blk, 16), lambda g: (g, 0, 0)),
            pl.BlockSpec((1, blk, 16), lambda g: (g, 0, 0)),
            idx_spec, idx_spec, idx_spec, idx_spec,
        ],
        out_specs=[
            pl.BlockSpec((1, 1, blk), lambda g: (g, 0, 0)),
            pl.BlockSpec(memory_space=pltpu.SMEM),
        ],
        out_shape=[
            jax.ShapeDtypeStruct((nblk, 1, blk), jnp.float32),
            jax.ShapeDtypeStruct((4,), jnp.float32),
        ],
    )(posd, negd, e0r, e1r, n0r, n1r)


def _pad_edges(idx, ntiles, group, even=False):
    k = idx.shape[0]
    per = -(-k // (ntiles * group))
    if even:
        per += per % 2
    tot = ntiles * per * group
    pad = jnp.full((tot - k,), NPAD - 1, jnp.int32)
    return jnp.concatenate([idx, pad]).reshape(ntiles, per, group)


def kernel(features, edge_index, neg_edge_index, W1, b1, W2, b2):
    e = edge_index.shape[1]
    src = edge_index[0].astype(jnp.int32)
    dst = edge_index[1].astype(jnp.int32)

    srcp = _pad_edges(src, NS, GB, even=True)
    dstp = _pad_edges(dst, NS, GB, even=True)
    dstp2 = _pad_edges(dst, NC * NS, GB)
    zrows = jnp.zeros((RPT, 128), jnp.float32)
    orows = jnp.ones((GB, 128), jnp.float32)
    hist = _hist(dstp2, zrows, orows)

    fpad = jnp.zeros((NPAD, DD), jnp.float32).at[:NN].set(features)
    xw1h = _mm_scale(fpad, W1, b1.reshape(1, DD), hist, relu_in=False)

    ps1 = _segsum(xw1h, srcp, dstp)
    xw2h = _mm_scale(ps1, W2, b2.reshape(1, DD), hist, relu_in=True)
    ps2 = _segsum(xw2h, srcp, dstp)

    repspad = _normalize(ps2, hist)

    ea = _pad_edges(jnp.concatenate([src, neg_edge_index[0].astype(jnp.int32)]),
                    NC * NS, 64, even=True)
    eb = _pad_edges(jnp.concatenate([dst, neg_edge_index[1].astype(jnp.int32)]),
                    NC * NS, 64, even=True)
    dots = _edge_dots(repspad, ea, eb)

    nblk = 80
    blk = e // nblk
    posd = dots[:e].reshape(nblk, blk, 16)
    negd = dots[e:2 * e].reshape(nblk, blk, 16)
    e0r = src.reshape(nblk, 1, blk)
    e1r = dst.reshape(nblk, 1, blk)
    n0r = neg_edge_index[0].astype(jnp.int32).reshape(nblk, 1, blk)
    n1r = neg_edge_index[1].astype(jnp.int32).reshape(nblk, 1, blk)
    est3, accs = _loss(posd, negd, e0r, e1r, n0r, n1r)

    reps = repspad[:NN]
    est = est3.reshape(e)
    rec_loss = (accs[1] + accs[0]) * NN / (accs[2] + accs[3])
    return reps, rec_loss, est


# speedup vs baseline: 5.0237x; 1.2689x over previous
import functools

import jax
import jax.numpy as jnp
from jax import lax
from jax.experimental import pallas as pl
from jax.experimental.pallas import tpu as pltpu
from jax.experimental.pallas import tpu_sc as plsc

NN = 10000
DD = 256
TAU = 0.1
NPAD = 10240
NC, NS = 2, 16
RPT = NPAD // NS
GB = 128


def _sc_mesh():
    return plsc.VectorSubcoreMesh(core_axis_name="c", subcore_axis_name="s")


def _hist(dstp, zrows, orows):
    g = dstp.shape[1]

    @functools.partial(
        pl.kernel,
        out_type=jax.ShapeDtypeStruct((NC, NPAD, 128), jnp.float32),
        mesh=_sc_mesh(),
        scratch_types=[
            pltpu.VMEM_SHARED((NPAD, 128), jnp.float32),
            pltpu.VMEM((g, 128), jnp.int32),
            pltpu.VMEM((GB, 128), jnp.float32),
        ],
    )
    def k(dst_hbm, z_hbm, o_hbm, out_hbm, acc, idst, ones_v):
        c = lax.axis_index("c")
        s = lax.axis_index("s")
        w = c * NS + s
        pltpu.sync_copy(z_hbm, acc.at[pl.ds(s * RPT, RPT)])
        pltpu.sync_copy(o_hbm, ones_v)
        pltpu.sync_copy(dst_hbm.at[w], idst)
        plsc.subcore_barrier()

        def body(j, carry):
            pltpu.sync_copy(ones_v, acc.at[idst.at[j]], add=True)
            return carry

        lax.fori_loop(0, g, body, 0)
        plsc.subcore_barrier()
        pltpu.sync_copy(acc.at[pl.ds(s * RPT, RPT)],
                        out_hbm.at[c, pl.ds(s * RPT, RPT)])

    return k(dstp, zrows, orows)


def _segsum(xwh, srcp, dstp):
    g, gb = srcp.shape[1], srcp.shape[2]
    ch_g = 16
    assert g % ch_g == 0 and ch_g % 8 == 0

    @functools.partial(
        pl.kernel,
        out_type=jax.ShapeDtypeStruct((NC, NPAD, 128), jnp.float32),
        mesh=_sc_mesh(),
        scratch_types=[
            pltpu.VMEM_SHARED((NPAD, 128), jnp.float32),
            pltpu.VMEM((ch_g, gb), jnp.int32),
            pltpu.VMEM((ch_g, gb), jnp.int32),
            pltpu.VMEM((gb, 128), jnp.float32),
            pltpu.VMEM((gb, 128), jnp.float32),
            pltpu.SemaphoreType.DMA,
            pltpu.SemaphoreType.DMA,
        ],
    )
    def k(xwh_hbm, src_hbm, dst_hbm, out_hbm, acc, isrc, idst, rowsa, rowsb,
          sema, semb):
        c = lax.axis_index("c")
        s = lax.axis_index("s")
        pltpu.sync_copy(xwh_hbm.at[c, pl.ds(s * RPT, RPT)],
                        acc.at[pl.ds(s * RPT, RPT)])
        plsc.subcore_barrier()
        xwc = xwh_hbm.at[c]

        def chunk(ci, carry):
            pltpu.sync_copy(src_hbm.at[s, pl.ds(ci * ch_g, ch_g)], isrc)
            pltpu.sync_copy(dst_hbm.at[s, pl.ds(ci * ch_g, ch_g)], idst)
            pltpu.async_copy(xwc.at[isrc.at[0]], rowsa, sema)
            pltpu.async_copy(xwc.at[isrc.at[1]], rowsb, semb)

            def body(i, c2):
                j = 2 * i
                pltpu.make_async_copy(xwc.at[isrc.at[0]], rowsa, sema).wait()
                pltpu.sync_copy(rowsa, acc.at[idst.at[j]], add=True)

                @pl.when(j + 2 < ch_g)
                def _():
                    pltpu.async_copy(xwc.at[isrc.at[j + 2]], rowsa, sema)

                pltpu.make_async_copy(xwc.at[isrc.at[0]], rowsb, semb).wait()
                pltpu.sync_copy(rowsb, acc.at[idst.at[j + 1]], add=True)

                @pl.when(j + 3 < ch_g)
                def _():
                    pltpu.async_copy(xwc.at[isrc.at[j + 3]], rowsb, semb)

                return c2

            lax.fori_loop(0, ch_g // 2, body, 0)
            return carry

        lax.fori_loop(0, g // ch_g, chunk, 0)
        plsc.subcore_barrier()
        pltpu.sync_copy(acc.at[pl.ds(s * RPT, RPT)],
                        out_hbm.at[c, pl.ds(s * RPT, RPT)])

    return k(xwh, srcp, dstp)


def _edge_dots(reps2, ea, eb):
    g3, gb = ea.shape[1], ea.shape[2]
    eout = NC * NS * g3 * gb

    @functools.partial(
        pl.kernel,
        out_type=jax.ShapeDtypeStruct((eout,), jnp.float32),
        mesh=_sc_mesh(),
        scratch_types=[
            pltpu.VMEM((g3, gb), jnp.int32),
            pltpu.VMEM((g3, gb), jnp.int32),
            pltpu.VMEM((gb, 256), jnp.float32),
            pltpu.VMEM((gb, 256), jnp.float32),
            pltpu.VMEM((gb, 256), jnp.float32),
            pltpu.VMEM((gb, 256), jnp.float32),
            pltpu.VMEM((gb,), jnp.float32),
            pltpu.VMEM((gb,), jnp.float32),
            pltpu.SemaphoreType.DMA,
            pltpu.SemaphoreType.DMA,
        ],
    )
    def k(reps_hbm, ea_hbm, eb_hbm, out_hbm, ia, ib, raa, rba, rab, rbb,
          dva, dvb, sema, semb):
        c = lax.axis_index("c")
        s = lax.axis_index("s")
        w = c * NS + s
        rc = reps_hbm.at[c]
        pltpu.sync_copy(ea_hbm.at[w], ia)
        pltpu.sync_copy(eb_hbm.at[w], ib)
        pltpu.async_copy(rc.at[ia.at[0]], raa, sema)
        pltpu.async_copy(rc.at[ib.at[0]], rba, sema)
        pltpu.async_copy(rc.at[ia.at[1]], rab, semb)
        pltpu.async_copy(rc.at[ib.at[1]], rbb, semb)

        lanes = lax.iota(jnp.int32, 16)
        _dn = lax.GatherDimensionNumbers(
            offset_dims=(), collapsed_slice_dims=(0,), start_index_map=(0,))

        def _shuf(x, perm):
            return lax.gather(x, perm[:, None], _dn, (1,),
                              mode=lax.GatherScatterMode.PROMISE_IN_BOUNDS)

        def dot_group(ra, rb, dv):
            def sub(sb, carry):
                def edge16(t, vec):
                    e = sb * 16 + t
                    acc = ra[e, pl.ds(0, 16)] * rb[e, pl.ds(0, 16)]
                    for v in range(1, 16):
                        acc = acc + ra[e, pl.ds(v * 16, 16)] * rb[e, pl.ds(v * 16, 16)]
                    for h in (8, 4, 2, 1):
                        acc = acc + _shuf(acc, lanes ^ h)
                    tv = jnp.full((16,), t, jnp.int32)
                    return jnp.where(lanes == tv, acc, vec)

                vec = lax.fori_loop(0, 16, edge16, jnp.zeros((16,), jnp.float32))
                dv[pl.ds(sb * 16, 16)] = vec
                return carry

            lax.fori_loop(0, gb // 16, sub, 0)

        def body(i, carry):
            j = 2 * i
            pltpu.make_async_copy(rc.at[ia.at[0]], raa, sema).wait()
            pltpu.make_async_copy(rc.at[ia.at[0]], rba, sema).wait()
            dot_group(raa, rba, dva)

            @pl.when(j + 2 < g3)
            def _():
                pltpu.async_copy(rc.at[ia.at[j + 2]], raa, sema)
                pltpu.async_copy(rc.at[ib.at[j + 2]], rba, sema)

            pltpu.sync_copy(dva, out_hbm.at[pl.ds((w * g3 + j) * gb, gb)])

            pltpu.make_async_copy(rc.at[ia.at[0]], rab, semb).wait()
            pltpu.make_async_copy(rc.at[ia.at[0]], rbb, semb).wait()
            dot_group(rab, rbb, dvb)

            @pl.when(j + 3 < g3)
            def _():
                pltpu.async_copy(rc.at[ia.at[j + 3]], rab, semb)
                pltpu.async_copy(rc.at[ib.at[j + 3]], rbb, semb)

            pltpu.sync_copy(dvb, out_hbm.at[pl.ds((w * g3 + j + 1) * gb, gb)])
            return carry

        lax.fori_loop(0, g3 // 2, body, 0)

    return k(reps2, ea, eb)


def _dinv_block(hist_ref, i):
    deg = hist_ref[0, :, 0:1] + hist_ref[1, :, 0:1] + 1.0
    row = lax.broadcasted_iota(jnp.int32, (256, 1), 0) + i * 256
    return jnp.where(row < NN, lax.rsqrt(deg), 0.0)


def _mm_scale(x, w, b2d, hist, relu_in):
    nblk = NPAD // 256

    def body(x_ref, w_ref, b_ref, h_ref, out_ref):
        i = pl.program_id(0)
        dv = _dinv_block(h_ref, i)
        if relu_in:
            xin = jnp.concatenate([x_ref[0], x_ref[1]], axis=1)
            xin = jnp.maximum(xin * dv, 0.0)
        else:
            xin = x_ref[...]
        xw = jnp.dot(xin, w_ref[...], preferred_element_type=jnp.float32)
        s = (xw + b_ref[...]) * dv
        out_ref[0] = s[:, :128]
        out_ref[1] = s[:, 128:]

    in_spec0 = (pl.BlockSpec((NC, 256, 128), lambda i: (0, i, 0)) if relu_in
                else pl.BlockSpec((256, 256), lambda i: (i, 0)))
    return pl.pallas_call(
        body,
        grid=(nblk,),
        in_specs=[
            in_spec0,
            pl.BlockSpec((256, 256), lambda i: (0, 0)),
            pl.BlockSpec((1, 256), lambda i: (0, 0)),
            pl.BlockSpec((NC, 256, 128), lambda i: (0, i, 0)),
        ],
        out_specs=pl.BlockSpec((NC, 256, 128), lambda i: (0, i, 0)),
        out_shape=jax.ShapeDtypeStruct((NC, NPAD, 128), jnp.float32),
    )(x, w, b2d, hist)


def _normalize(presum, hist):
    nblk = NPAD // 256

    def body(p_ref, h_ref, out_ref):
        i = pl.program_id(0)
        dv = _dinv_block(h_ref, i)
        r = jnp.concatenate([p_ref[0], p_ref[1]], axis=1) * dv
        nrm = jnp.sqrt(jnp.sum(r * r, axis=1, keepdims=True))
        rn = r / jnp.maximum(nrm, 1e-12)
        out_ref[0] = rn
        out_ref[1] = rn

    return pl.pallas_call(
        body,
        grid=(nblk,),
        in_specs=[
            pl.BlockSpec((NC, 256, 128), lambda i: (0, i, 0)),
            pl.BlockSpec((NC, 256, 128), lambda i: (0, i, 0)),
        ],
        out_specs=pl.BlockSpec((2, 256, 256), lambda i: (0, i, 0)),
        out_shape=jax.ShapeDtypeStruct((2, NPAD, 256), jnp.float32),
    )(presum, hist)


def _loss(posd, negd, e0r, e1r, n0r, n1r):
    nblk, blk = posd.shape[0], posd.shape[2]

    def body(pd_ref, nd_ref, e0_ref, e1_ref, n0_ref, n1_ref, est_ref, acc_ref):
        g = pl.program_id(0)
        pd = pd_ref[0, 0]
        nd = nd_ref[0, 0]
        pm = (e0_ref[0, 0] < e1_ref[0, 0]).astype(jnp.float32)
        nm = (n0_ref[0, 0] < n1_ref[0, 0]).astype(jnp.float32)
        est = jnp.maximum(pd, 0.0)
        est_ref[0, 0] = jnp.where(est < TAU, 0.0, est)

        @pl.when(g == 0)
        def _():
            acc_ref[0] = 0.0
            acc_ref[1] = 0.0
            acc_ref[2] = 0.0
            acc_ref[3] = 0.0

        acc_ref[0] += jnp.sum(pm * (pd - 1.0) ** 2)
        acc_ref[1] += jnp.sum(nm * nd * nd)
        acc_ref[2] += jnp.sum(pm)
        acc_ref[3] += jnp.sum(nm)

    idx_spec = pl.BlockSpec((1, 1, blk), lambda g: (g, 0, 0))
    return pl.pallas_call(
        body,
        grid=(nblk,),
        in_specs=[idx_spec, idx_spec, idx_spec, idx_spec, idx_spec, idx_spec],
        out_specs=[
            pl.BlockSpec((1, 1, blk), lambda g: (g, 0, 0)),
            pl.BlockSpec(memory_space=pltpu.SMEM),
        ],
        out_shape=[
            jax.ShapeDtypeStruct((nblk, 1, blk), jnp.float32),
            jax.ShapeDtypeStruct((4,), jnp.float32),
        ],
    )(posd, negd, e0r, e1r, n0r, n1r)


def _pad_edges(idx, ntiles, group, even=False):
    k = idx.shape[0]
    per = -(-k // (ntiles * group))
    if even:
        per += per % 2
    tot = ntiles * per * group
    pad = jnp.full((tot - k,), NPAD - 1, jnp.int32)
    return jnp.concatenate([idx, pad]).reshape(ntiles, per, group)


def kernel(features, edge_index, neg_edge_index, W1, b1, W2, b2):
    e = edge_index.shape[1]
    src = edge_index[0].astype(jnp.int32)
    dst = edge_index[1].astype(jnp.int32)

    srcp = _pad_edges(src, NS, GB, even=True)
    dstp = _pad_edges(dst, NS, GB, even=True)
    dstp2 = _pad_edges(dst, NC * NS, GB)
    zrows = jnp.zeros((RPT, 128), jnp.float32)
    orows = jnp.ones((GB, 128), jnp.float32)
    hist = _hist(dstp2, zrows, orows)

    fpad = jnp.zeros((NPAD, DD), jnp.float32).at[:NN].set(features)
    xw1h = _mm_scale(fpad, W1, b1.reshape(1, DD), hist, relu_in=False)

    ps1 = _segsum(xw1h, srcp, dstp)
    xw2h = _mm_scale(ps1, W2, b2.reshape(1, DD), hist, relu_in=True)
    ps2 = _segsum(xw2h, srcp, dstp)

    reps2 = _normalize(ps2, hist)

    ea = _pad_edges(jnp.concatenate([src, neg_edge_index[0].astype(jnp.int32)]),
                    NC * NS, 64, even=True)
    eb = _pad_edges(jnp.concatenate([dst, neg_edge_index[1].astype(jnp.int32)]),
                    NC * NS, 64, even=True)
    dots = _edge_dots(reps2, ea, eb)

    nblk = 20
    blk = e // nblk
    posd = dots[:e].reshape(nblk, 1, blk)
    negd = dots[e:2 * e].reshape(nblk, 1, blk)
    e0r = src.reshape(nblk, 1, blk)
    e1r = dst.reshape(nblk, 1, blk)
    n0r = neg_edge_index[0].astype(jnp.int32).reshape(nblk, 1, blk)
    n1r = neg_edge_index[1].astype(jnp.int32).reshape(nblk, 1, blk)
    est3, accs = _loss(posd, negd, e0r, e1r, n0r, n1r)

    reps = reps2[0, :NN]
    est = est3.reshape(e)
    rec_loss = (accs[1] + accs[0]) * NN / (accs[2] + accs[3])
    return reps, rec_loss, est


# speedup vs baseline: 5.5992x; 1.1145x over previous
import functools

import jax
import jax.numpy as jnp
from jax import lax
from jax.experimental import pallas as pl
from jax.experimental.pallas import tpu as pltpu
from jax.experimental.pallas import tpu_sc as plsc

NN = 10000
DD = 256
TAU = 0.1
NPAD = 10240
NC, NS = 2, 16
RPT = NPAD // NS
GB = 128


def _sc_mesh():
    return plsc.VectorSubcoreMesh(core_axis_name="c", subcore_axis_name="s")


def _hist(dstp, zrows, orows):
    g = dstp.shape[1]

    @functools.partial(
        pl.kernel,
        out_type=jax.ShapeDtypeStruct((NC, NPAD, 128), jnp.float32),
        mesh=_sc_mesh(),
        scratch_types=[
            pltpu.VMEM_SHARED((NPAD, 128), jnp.float32),
            pltpu.VMEM((g, 128), jnp.int32),
            pltpu.VMEM((GB, 128), jnp.float32),
        ],
    )
    def k(dst_hbm, z_hbm, o_hbm, out_hbm, acc, idst, ones_v):
        c = lax.axis_index("c")
        s = lax.axis_index("s")
        w = c * NS + s
        pltpu.sync_copy(z_hbm, acc.at[pl.ds(s * RPT, RPT)])
        pltpu.sync_copy(o_hbm, ones_v)
        pltpu.sync_copy(dst_hbm.at[w], idst)
        plsc.subcore_barrier()

        def body(j, carry):
            pltpu.sync_copy(ones_v, acc.at[idst.at[j]], add=True)
            return carry

        lax.fori_loop(0, g, body, 0)
        plsc.subcore_barrier()
        pltpu.sync_copy(acc.at[pl.ds(s * RPT, RPT)],
                        out_hbm.at[c, pl.ds(s * RPT, RPT)])

    return k(dstp, zrows, orows)


def _segsum(xwh, srcp, dstp):
    g, gb = srcp.shape[1], srcp.shape[2]
    ch_g = 16
    assert g % ch_g == 0 and ch_g % 8 == 0

    @functools.partial(
        pl.kernel,
        out_type=jax.ShapeDtypeStruct((NC, NPAD, 128), jnp.float32),
        mesh=_sc_mesh(),
        scratch_types=[
            pltpu.VMEM_SHARED((NPAD, 128), jnp.float32),
            pltpu.VMEM((ch_g, gb), jnp.int32),
            pltpu.VMEM((ch_g, gb), jnp.int32),
            pltpu.VMEM((gb, 128), jnp.float32),
            pltpu.VMEM((gb, 128), jnp.float32),
            pltpu.SemaphoreType.DMA,
            pltpu.SemaphoreType.DMA,
        ],
    )
    def k(xwh_hbm, src_hbm, dst_hbm, out_hbm, acc, isrc, idst, rowsa, rowsb,
          sema, semb):
        c = lax.axis_index("c")
        s = lax.axis_index("s")
        pltpu.sync_copy(xwh_hbm.at[c, pl.ds(s * RPT, RPT)],
                        acc.at[pl.ds(s * RPT, RPT)])
        plsc.subcore_barrier()
        xwc = xwh_hbm.at[c]

        def chunk(ci, carry):
            pltpu.sync_copy(src_hbm.at[s, pl.ds(ci * ch_g, ch_g)], isrc)
            pltpu.sync_copy(dst_hbm.at[s, pl.ds(ci * ch_g, ch_g)], idst)
            pltpu.async_copy(xwc.at[isrc.at[0]], rowsa, sema)
            pltpu.async_copy(xwc.at[isrc.at[1]], rowsb, semb)

            def body(i, c2):
                j = 2 * i
                pltpu.make_async_copy(xwc.at[isrc.at[0]], rowsa, sema).wait()
                pltpu.sync_copy(rowsa, acc.at[idst.at[j]], add=True)

                @pl.when(j + 2 < ch_g)
                def _():
                    pltpu.async_copy(xwc.at[isrc.at[j + 2]], rowsa, sema)

                pltpu.make_async_copy(xwc.at[isrc.at[0]], rowsb, semb).wait()
                pltpu.sync_copy(rowsb, acc.at[idst.at[j + 1]], add=True)

                @pl.when(j + 3 < ch_g)
                def _():
                    pltpu.async_copy(xwc.at[isrc.at[j + 3]], rowsb, semb)

                return c2

            lax.fori_loop(0, ch_g // 2, body, 0)
            return carry

        lax.fori_loop(0, g // ch_g, chunk, 0)
        plsc.subcore_barrier()
        pltpu.sync_copy(acc.at[pl.ds(s * RPT, RPT)],
                        out_hbm.at[c, pl.ds(s * RPT, RPT)])

    return k(xwh, srcp, dstp)


def _edge_dots(reps2, ea, eb):
    g3, gb = ea.shape[1], ea.shape[2]
    eout = NC * NS * g3 * gb

    @functools.partial(
        pl.kernel,
        out_type=jax.ShapeDtypeStruct((eout,), jnp.float32),
        mesh=_sc_mesh(),
        scratch_types=[
            pltpu.VMEM((g3, gb), jnp.int32),
            pltpu.VMEM((g3, gb), jnp.int32),
            pltpu.VMEM((gb, 256), jnp.float32),
            pltpu.VMEM((gb, 256), jnp.float32),
            pltpu.VMEM((gb, 256), jnp.float32),
            pltpu.VMEM((gb, 256), jnp.float32),
            pltpu.VMEM((gb,), jnp.float32),
            pltpu.VMEM((gb,), jnp.float32),
            pltpu.SemaphoreType.DMA,
            pltpu.SemaphoreType.DMA,
        ],
    )
    def k(reps_hbm, ea_hbm, eb_hbm, out_hbm, ia, ib, raa, rba, rab, rbb,
          dva, dvb, sema, semb):
        c = lax.axis_index("c")
        s = lax.axis_index("s")
        w = c * NS + s
        rc = reps_hbm.at[c]
        pltpu.sync_copy(ea_hbm.at[w], ia)
        pltpu.sync_copy(eb_hbm.at[w], ib)
        pltpu.async_copy(rc.at[ia.at[0]], raa, sema)
        pltpu.async_copy(rc.at[ib.at[0]], rba, sema)
        pltpu.async_copy(rc.at[ia.at[1]], rab, semb)
        pltpu.async_copy(rc.at[ib.at[1]], rbb, semb)

        lanes = lax.iota(jnp.int32, 16)
        _dn = lax.GatherDimensionNumbers(
            offset_dims=(), collapsed_slice_dims=(0,), start_index_map=(0,))

        def _shuf(x, perm):
            return lax.gather(x, perm[:, None], _dn, (1,),
                              mode=lax.GatherScatterMode.PROMISE_IN_BOUNDS)

        def dot_group(ra, rb, dv):
            def sub(sb, carry):
                def edge16(t, vec):
                    e = sb * 16 + t
                    acc = ra[e, pl.ds(0, 16)] * rb[e, pl.ds(0, 16)]
                    for v in range(1, 16):
                        acc = acc + ra[e, pl.ds(v * 16, 16)] * rb[e, pl.ds(v * 16, 16)]
                    for h in (8, 4, 2, 1):
                        acc = acc + _shuf(acc, lanes ^ h)
                    tv = jnp.full((16,), t, jnp.int32)
                    return jnp.where(lanes == tv, acc, vec)

                vec = lax.fori_loop(0, 16, edge16, jnp.zeros((16,), jnp.float32))
                dv[pl.ds(sb * 16, 16)] = vec
                return carry

            lax.fori_loop(0, gb // 16, sub, 0)

        def body(i, carry):
            j = 2 * i
            pltpu.make_async_copy(rc.at[ia.at[0]], raa, sema).wait()
            pltpu.make_async_copy(rc.at[ia.at[0]], rba, sema).wait()
            dot_group(raa, rba, dva)

            @pl.when(j + 2 < g3)
            def _():
                pltpu.async_copy(rc.at[ia.at[j + 2]], raa, sema)
                pltpu.async_copy(rc.at[ib.at[j + 2]], rba, sema)

            pltpu.sync_copy(dva, out_hbm.at[pl.ds((w * g3 + j) * gb, gb)])

            pltpu.make_async_copy(rc.at[ia.at[0]], rab, semb).wait()
            pltpu.make_async_copy(rc.at[ia.at[0]], rbb, semb).wait()
            dot_group(rab, rbb, dvb)

            @pl.when(j + 3 < g3)
            def _():
                pltpu.async_copy(rc.at[ia.at[j + 3]], rab, semb)
                pltpu.async_copy(rc.at[ib.at[j + 3]], rbb, semb)

            pltpu.sync_copy(dvb, out_hbm.at[pl.ds((w * g3 + j + 1) * gb, gb)])
            return carry

        lax.fori_loop(0, g3 // 2, body, 0)

    return k(reps2, ea, eb)


def _dinv_block(hist_ref, i):
    deg = hist_ref[0, :, 0:1] + hist_ref[1, :, 0:1] + 1.0
    row = lax.broadcasted_iota(jnp.int32, (256, 1), 0) + i * 256
    return jnp.where(row < NN, lax.rsqrt(deg), 0.0)


def _mm_scale(x, w, b2d, hist, relu_in):
    nblk = NPAD // 256

    def body(x_ref, w_ref, b_ref, h_ref, out_ref):
        i = pl.program_id(0)
        dv = _dinv_block(h_ref, i)
        if relu_in:
            xin = jnp.concatenate([x_ref[0], x_ref[1]], axis=1)
            xin = jnp.maximum(xin * dv, 0.0)
        else:
            xin = x_ref[...]
        xw = jnp.dot(xin, w_ref[...], preferred_element_type=jnp.float32)
        s = (xw + b_ref[...]) * dv
        out_ref[0] = s[:, :128]
        out_ref[1] = s[:, 128:]

    in_spec0 = (pl.BlockSpec((NC, 256, 128), lambda i: (0, i, 0)) if relu_in
                else pl.BlockSpec((256, 256), lambda i: (i, 0)))
    return pl.pallas_call(
        body,
        grid=(nblk,),
        in_specs=[
            in_spec0,
            pl.BlockSpec((256, 256), lambda i: (0, 0)),
            pl.BlockSpec((1, 256), lambda i: (0, 0)),
            pl.BlockSpec((NC, 256, 128), lambda i: (0, i, 0)),
        ],
        out_specs=pl.BlockSpec((NC, 256, 128), lambda i: (0, i, 0)),
        out_shape=jax.ShapeDtypeStruct((NC, NPAD, 128), jnp.float32),
    )(x, w, b2d, hist)


def _normalize(presum, hist):
    nblk = NPAD // 256

    def body(p_ref, h_ref, out_ref):
        i = pl.program_id(0)
        dv = _dinv_block(h_ref, i)
        r = jnp.concatenate([p_ref[0], p_ref[1]], axis=1) * dv
        nrm = jnp.sqrt(jnp.sum(r * r, axis=1, keepdims=True))
        rn = r / jnp.maximum(nrm, 1e-12)
        out_ref[0] = rn
        out_ref[1] = rn

    return pl.pallas_call(
        body,
        grid=(nblk,),
        in_specs=[
            pl.BlockSpec((NC, 256, 128), lambda i: (0, i, 0)),
            pl.BlockSpec((NC, 256, 128), lambda i: (0, i, 0)),
        ],
        out_specs=pl.BlockSpec((2, 256, 256), lambda i: (0, i, 0)),
        out_shape=jax.ShapeDtypeStruct((2, NPAD, 256), jnp.float32),
    )(presum, hist)


def _loss(posd, negd, e0r, e1r, n0r, n1r):
    nblk, blk = posd.shape[0], posd.shape[2]

    def body(pd_ref, nd_ref, e0_ref, e1_ref, n0_ref, n1_ref, est_ref, acc_ref):
        g = pl.program_id(0)
        pd = pd_ref[0, 0]
        nd = nd_ref[0, 0]
        pm = (e0_ref[0, 0] < e1_ref[0, 0]).astype(jnp.float32)
        nm = (n0_ref[0, 0] < n1_ref[0, 0]).astype(jnp.float32)
        est = jnp.maximum(pd, 0.0)
        est_ref[0, 0] = jnp.where(est < TAU, 0.0, est)

        @pl.when(g == 0)
        def _():
            acc_ref[0] = 0.0
            acc_ref[1] = 0.0
            acc_ref[2] = 0.0
            acc_ref[3] = 0.0

        acc_ref[0] += jnp.sum(pm * (pd - 1.0) ** 2)
        acc_ref[1] += jnp.sum(nm * nd * nd)
        acc_ref[2] += jnp.sum(pm)
        acc_ref[3] += jnp.sum(nm)

    idx_spec = pl.BlockSpec((1, 1, blk), lambda g: (g, 0, 0))
    return pl.pallas_call(
        body,
        grid=(nblk,),
        in_specs=[idx_spec, idx_spec, idx_spec, idx_spec, idx_spec, idx_spec],
        out_specs=[
            pl.BlockSpec((1, 1, blk), lambda g: (g, 0, 0)),
            pl.BlockSpec(memory_space=pltpu.SMEM),
        ],
        out_shape=[
            jax.ShapeDtypeStruct((nblk, 1, blk), jnp.float32),
            jax.ShapeDtypeStruct((4,), jnp.float32),
        ],
    )(posd, negd, e0r, e1r, n0r, n1r)


def _pad_edges(idx, ntiles, group, even=False):
    k = idx.shape[0]
    per = -(-k // (ntiles * group))
    if even:
        per += per % 2
    tot = ntiles * per * group
    pad = jnp.full((tot - k,), NPAD - 1, jnp.int32)
    return jnp.concatenate([idx, pad]).reshape(ntiles, per, group)


def kernel(features, edge_index, neg_edge_index, W1, b1, W2, b2):
    e = edge_index.shape[1]
    src = edge_index[0].astype(jnp.int32)
    dst = edge_index[1].astype(jnp.int32)

    srcp = _pad_edges(src, NS, GB, even=True)
    dstp = _pad_edges(dst, NS, GB, even=True)
    dstp2 = _pad_edges(dst, NC * NS, GB)
    zrows = jnp.zeros((RPT, 128), jnp.float32)
    orows = jnp.ones((GB, 128), jnp.float32)
    hist = _hist(dstp2, zrows, orows)

    fpad = jnp.zeros((NPAD, DD), jnp.float32).at[:NN].set(features)
    xw1h = _mm_scale(fpad, W1, b1.reshape(1, DD), hist, relu_in=False)

    ps1 = _segsum(xw1h, srcp, dstp)
    xw2h = _mm_scale(ps1, W2, b2.reshape(1, DD), hist, relu_in=True)
    ps2 = _segsum(xw2h, srcp, dstp)

    reps2 = _normalize(ps2, hist)

    nw = NC * NS
    ept = e // nw
    gp = -(-ept // 64)
    gp += gp % 2
    eptp = gp * 64

    def tile_pad(idx):
        blocks = idx.reshape(nw, ept)
        fill = jnp.full((nw, eptp - ept), NPAD - 1, jnp.int32)
        return jnp.concatenate([blocks, fill], axis=1).reshape(nw, gp, 64)

    n0 = neg_edge_index[0].astype(jnp.int32)
    n1 = neg_edge_index[1].astype(jnp.int32)
    ea = jnp.concatenate([tile_pad(src), tile_pad(n0)], axis=1)
    eb = jnp.concatenate([tile_pad(dst), tile_pad(n1)], axis=1)
    dots = _edge_dots(reps2, ea, eb)

    nblk = 20
    blk = e // nblk
    dview = dots.reshape(nw, 2 * gp, 64)
    posd = dview[:, :gp].reshape(nw, eptp)[:, :ept].reshape(nblk, 1, blk)
    negd = dview[:, gp:].reshape(nw, eptp)[:, :ept].reshape(nblk, 1, blk)
    e0r = src.reshape(nblk, 1, blk)
    e1r = dst.reshape(nblk, 1, blk)
    n0r = neg_edge_index[0].astype(jnp.int32).reshape(nblk, 1, blk)
    n1r = neg_edge_index[1].astype(jnp.int32).reshape(nblk, 1, blk)
    est3, accs = _loss(posd, negd, e0r, e1r, n0r, n1r)

    reps = reps2[0, :NN]
    est = est3.reshape(e)
    rec_loss = (accs[1] + accs[0]) * NN / (accs[2] + accs[3])
    return reps, rec_loss, est


# speedup vs baseline: 6.0582x; 1.0820x over previous
import functools

import jax
import jax.numpy as jnp
from jax import lax
from jax.experimental import pallas as pl
from jax.experimental.pallas import tpu as pltpu
from jax.experimental.pallas import tpu_sc as plsc

NN = 10000
DD = 256
TAU = 0.1
NPAD = 10240
NC, NS = 2, 16
RPT = NPAD // NS
GB = 128


def _sc_mesh():
    return plsc.VectorSubcoreMesh(core_axis_name="c", subcore_axis_name="s")


def _hist(dstp, zrows, orows):
    g = dstp.shape[1]

    @functools.partial(
        pl.kernel,
        out_type=jax.ShapeDtypeStruct((NC, NPAD, 128), jnp.float32),
        mesh=_sc_mesh(),
        scratch_types=[
            pltpu.VMEM_SHARED((NPAD, 128), jnp.float32),
            pltpu.VMEM((g, 128), jnp.int32),
            pltpu.VMEM((GB, 128), jnp.float32),
        ],
    )
    def k(dst_hbm, z_hbm, o_hbm, out_hbm, acc, idst, ones_v):
        c = lax.axis_index("c")
        s = lax.axis_index("s")
        w = c * NS + s
        pltpu.sync_copy(z_hbm, acc.at[pl.ds(s * RPT, RPT)])
        pltpu.sync_copy(o_hbm, ones_v)
        pltpu.sync_copy(dst_hbm.at[w], idst)
        plsc.subcore_barrier()

        def body(j, carry):
            pltpu.sync_copy(ones_v, acc.at[idst.at[j]], add=True)
            return carry

        lax.fori_loop(0, g, body, 0)
        plsc.subcore_barrier()
        pltpu.sync_copy(acc.at[pl.ds(s * RPT, RPT)],
                        out_hbm.at[c, pl.ds(s * RPT, RPT)])

    return k(dstp, zrows, orows)


def _segsum(xwh, srcp, dstp):
    g, gb = srcp.shape[1], srcp.shape[2]

    @functools.partial(
        pl.kernel,
        out_type=jax.ShapeDtypeStruct((NC, NPAD, 128), jnp.float32),
        mesh=_sc_mesh(),
        scratch_types=[
            pltpu.VMEM_SHARED((NPAD, 128), jnp.float32),
            pltpu.VMEM((g, gb), jnp.int32),
            pltpu.VMEM((g, gb), jnp.int32),
            pltpu.VMEM((gb, 128), jnp.float32),
            pltpu.SemaphoreType.DMA,
        ],
    )
    def k(xwh_hbm, src_hbm, dst_hbm, out_hbm, acc, isrc, idst, rows, sem):
        c = lax.axis_index("c")
        s = lax.axis_index("s")
        pltpu.sync_copy(xwh_hbm.at[c, pl.ds(s * RPT, RPT)],
                        acc.at[pl.ds(s * RPT, RPT)])
        pltpu.sync_copy(src_hbm.at[s], isrc)
        pltpu.sync_copy(dst_hbm.at[s], idst)
        plsc.subcore_barrier()
        xwc = xwh_hbm.at[c]

        def body(j, carry):
            pltpu.async_copy(xwc.at[isrc.at[j]], rows, sem).wait()
            pltpu.sync_copy(rows, acc.at[idst.at[j]], add=True)
            return carry

        lax.fori_loop(0, g, body, 0)
        plsc.subcore_barrier()
        pltpu.sync_copy(acc.at[pl.ds(s * RPT, RPT)],
                        out_hbm.at[c, pl.ds(s * RPT, RPT)])

    return k(xwh, srcp, dstp)


def _edge_dots(reps2, ea, eb):
    g3, gb = ea.shape[1], ea.shape[2]
    eout = NC * NS * g3 * gb

    @functools.partial(
        pl.kernel,
        out_type=jax.ShapeDtypeStruct((eout,), jnp.float32),
        mesh=_sc_mesh(),
        scratch_types=[
            pltpu.VMEM((g3, gb), jnp.int32),
            pltpu.VMEM((g3, gb), jnp.int32),
            pltpu.VMEM((gb, 256), jnp.float32),
            pltpu.VMEM((gb, 256), jnp.float32),
            pltpu.VMEM((gb, 256), jnp.float32),
            pltpu.VMEM((gb, 256), jnp.float32),
            pltpu.VMEM((gb,), jnp.float32),
            pltpu.VMEM((gb,), jnp.float32),
            pltpu.SemaphoreType.DMA,
            pltpu.SemaphoreType.DMA,
        ],
    )
    def k(reps_hbm, ea_hbm, eb_hbm, out_hbm, ia, ib, raa, rba, rab, rbb,
          dva, dvb, sema, semb):
        c = lax.axis_index("c")
        s = lax.axis_index("s")
        w = c * NS + s
        rc = reps_hbm.at[c]
        pltpu.sync_copy(ea_hbm.at[w], ia)
        pltpu.sync_copy(eb_hbm.at[w], ib)
        pltpu.async_copy(rc.at[ia.at[0]], raa, sema)
        pltpu.async_copy(rc.at[ib.at[0]], rba, sema)
        pltpu.async_copy(rc.at[ia.at[1]], rab, semb)
        pltpu.async_copy(rc.at[ib.at[1]], rbb, semb)

        lanes = lax.iota(jnp.int32, 16)
        _dn = lax.GatherDimensionNumbers(
            offset_dims=(), collapsed_slice_dims=(0,), start_index_map=(0,))

        def _shuf(x, perm):
            return lax.gather(x, perm[:, None], _dn, (1,),
                              mode=lax.GatherScatterMode.PROMISE_IN_BOUNDS)

        def dot_group(ra, rb, dv):
            def sub(sb, carry):
                def edge16(t, vec):
                    e = sb * 16 + t
                    acc = ra[e, pl.ds(0, 16)] * rb[e, pl.ds(0, 16)]
                    for v in range(1, 16):
                        acc = acc + ra[e, pl.ds(v * 16, 16)] * rb[e, pl.ds(v * 16, 16)]
                    for h in (8, 4, 2, 1):
                        acc = acc + _shuf(acc, lanes ^ h)
                    tv = jnp.full((16,), t, jnp.int32)
                    return jnp.where(lanes == tv, acc, vec)

                vec = lax.fori_loop(0, 16, edge16, jnp.zeros((16,), jnp.float32))
                dv[pl.ds(sb * 16, 16)] = vec
                return carry

            lax.fori_loop(0, gb // 16, sub, 0)

        def body(i, carry):
            j = 2 * i
            pltpu.make_async_copy(rc.at[ia.at[0]], raa, sema).wait()
            pltpu.make_async_copy(rc.at[ia.at[0]], rba, sema).wait()
            dot_group(raa, rba, dva)

            @pl.when(j + 2 < g3)
            def _():
                pltpu.async_copy(rc.at[ia.at[j + 2]], raa, sema)
                pltpu.async_copy(rc.at[ib.at[j + 2]], rba, sema)

            pltpu.sync_copy(dva, out_hbm.at[pl.ds((w * g3 + j) * gb, gb)])

            pltpu.make_async_copy(rc.at[ia.at[0]], rab, semb).wait()
            pltpu.make_async_copy(rc.at[ia.at[0]], rbb, semb).wait()
            dot_group(rab, rbb, dvb)

            @pl.when(j + 3 < g3)
            def _():
                pltpu.async_copy(rc.at[ia.at[j + 3]], rab, semb)
                pltpu.async_copy(rc.at[ib.at[j + 3]], rbb, semb)

            pltpu.sync_copy(dvb, out_hbm.at[pl.ds((w * g3 + j + 1) * gb, gb)])
            return carry

        lax.fori_loop(0, g3 // 2, body, 0)

    return k(reps2, ea, eb)


def _dinv_block(hist_ref, i):
    deg = hist_ref[0, :, 0:1] + hist_ref[1, :, 0:1] + 1.0
    row = lax.broadcasted_iota(jnp.int32, (256, 1), 0) + i * 256
    return jnp.where(row < NN, lax.rsqrt(deg), 0.0)


def _mm_scale(x, w, b2d, hist, relu_in):
    nblk = NPAD // 256

    def body(x_ref, w_ref, b_ref, h_ref, out_ref):
        i = pl.program_id(0)
        dv = _dinv_block(h_ref, i)
        if relu_in:
            xin = jnp.concatenate([x_ref[0], x_ref[1]], axis=1)
            xin = jnp.maximum(xin * dv, 0.0)
        else:
            xin = x_ref[...]
        xw = jnp.dot(xin, w_ref[...], preferred_element_type=jnp.float32)
        s = (xw + b_ref[...]) * dv
        out_ref[0] = s[:, :128]
        out_ref[1] = s[:, 128:]

    in_spec0 = (pl.BlockSpec((NC, 256, 128), lambda i: (0, i, 0)) if relu_in
                else pl.BlockSpec((256, 256), lambda i: (i, 0)))
    return pl.pallas_call(
        body,
        grid=(nblk,),
        in_specs=[
            in_spec0,
            pl.BlockSpec((256, 256), lambda i: (0, 0)),
            pl.BlockSpec((1, 256), lambda i: (0, 0)),
            pl.BlockSpec((NC, 256, 128), lambda i: (0, i, 0)),
        ],
        out_specs=pl.BlockSpec((NC, 256, 128), lambda i: (0, i, 0)),
        out_shape=jax.ShapeDtypeStruct((NC, NPAD, 128), jnp.float32),
    )(x, w, b2d, hist)


def _normalize(presum, hist):
    nblk = NPAD // 256

    def body(p_ref, h_ref, out_ref):
        i = pl.program_id(0)
        dv = _dinv_block(h_ref, i)
        r = jnp.concatenate([p_ref[0], p_ref[1]], axis=1) * dv
        nrm = jnp.sqrt(jnp.sum(r * r, axis=1, keepdims=True))
        rn = r / jnp.maximum(nrm, 1e-12)
        out_ref[0] = rn
        out_ref[1] = rn

    return pl.pallas_call(
        body,
        grid=(nblk,),
        in_specs=[
            pl.BlockSpec((NC, 256, 128), lambda i: (0, i, 0)),
            pl.BlockSpec((NC, 256, 128), lambda i: (0, i, 0)),
        ],
        out_specs=pl.BlockSpec((2, 256, 256), lambda i: (0, i, 0)),
        out_shape=jax.ShapeDtypeStruct((2, NPAD, 256), jnp.float32),
    )(presum, hist)


def _loss(posd, negd, e0r, e1r, n0r, n1r):
    nblk, blk = posd.shape[0], posd.shape[2]

    def body(pd_ref, nd_ref, e0_ref, e1_ref, n0_ref, n1_ref, est_ref, acc_ref):
        g = pl.program_id(0)
        pd = pd_ref[0, 0]
        nd = nd_ref[0, 0]
        pm = (e0_ref[0, 0] < e1_ref[0, 0]).astype(jnp.float32)
        nm = (n0_ref[0, 0] < n1_ref[0, 0]).astype(jnp.float32)
        est = jnp.maximum(pd, 0.0)
        est_ref[0, 0] = jnp.where(est < TAU, 0.0, est)

        @pl.when(g == 0)
        def _():
            acc_ref[0] = 0.0
            acc_ref[1] = 0.0
            acc_ref[2] = 0.0
            acc_ref[3] = 0.0

        acc_ref[0] += jnp.sum(pm * (pd - 1.0) ** 2)
        acc_ref[1] += jnp.sum(nm * nd * nd)
        acc_ref[2] += jnp.sum(pm)
        acc_ref[3] += jnp.sum(nm)

    idx_spec = pl.BlockSpec((1, 1, blk), lambda g: (g, 0, 0))
    return pl.pallas_call(
        body,
        grid=(nblk,),
        in_specs=[idx_spec, idx_spec, idx_spec, idx_spec, idx_spec, idx_spec],
        out_specs=[
            pl.BlockSpec((1, 1, blk), lambda g: (g, 0, 0)),
            pl.BlockSpec(memory_space=pltpu.SMEM),
        ],
        out_shape=[
            jax.ShapeDtypeStruct((nblk, 1, blk), jnp.float32),
            jax.ShapeDtypeStruct((4,), jnp.float32),
        ],
    )(posd, negd, e0r, e1r, n0r, n1r)


def _pad_edges(idx, ntiles, group, even=False):
    k = idx.shape[0]
    per = -(-k // (ntiles * group))
    if even:
        per += per % 2
    tot = ntiles * per * group
    pad = jnp.full((tot - k,), NPAD - 1, jnp.int32)
    return jnp.concatenate([idx, pad]).reshape(ntiles, per, group)


def kernel(features, edge_index, neg_edge_index, W1, b1, W2, b2):
    e = edge_index.shape[1]
    src = edge_index[0].astype(jnp.int32)
    dst = edge_index[1].astype(jnp.int32)

    srcp = _pad_edges(src, NS, GB)
    dstp = _pad_edges(dst, NS, GB)
    dstp2 = _pad_edges(dst, NC * NS, GB)
    zrows = jnp.zeros((RPT, 128), jnp.float32)
    orows = jnp.ones((GB, 128), jnp.float32)
    hist = _hist(dstp2, zrows, orows)

    fpad = jnp.zeros((NPAD, DD), jnp.float32).at[:NN].set(features)
    xw1h = _mm_scale(fpad, W1, b1.reshape(1, DD), hist, relu_in=False)

    ps1 = _segsum(xw1h, srcp, dstp)
    xw2h = _mm_scale(ps1, W2, b2.reshape(1, DD), hist, relu_in=True)
    ps2 = _segsum(xw2h, srcp, dstp)

    reps2 = _normalize(ps2, hist)

    nw = NC * NS
    ept = e // nw
    gp = -(-ept // 64)
    gp += gp % 2
    eptp = gp * 64

    def tile_pad(idx):
        blocks = idx.reshape(nw, ept)
        fill = jnp.full((nw, eptp - ept), NPAD - 1, jnp.int32)
        return jnp.concatenate([blocks, fill], axis=1).reshape(nw, gp, 64)

    n0 = neg_edge_index[0].astype(jnp.int32)
    n1 = neg_edge_index[1].astype(jnp.int32)
    ea = jnp.concatenate([tile_pad(src), tile_pad(n0)], axis=1)
    eb = jnp.concatenate([tile_pad(dst), tile_pad(n1)], axis=1)
    dots = _edge_dots(reps2, ea, eb)

    nblk = 20
    blk = e // nblk
    dview = dots.reshape(nw, 2 * gp, 64)
    posd = dview[:, :gp].reshape(nw, eptp)[:, :ept].reshape(nblk, 1, blk)
    negd = dview[:, gp:].reshape(nw, eptp)[:, :ept].reshape(nblk, 1, blk)
    e0r = src.reshape(nblk, 1, blk)
    e1r = dst.reshape(nblk, 1, blk)
    n0r = neg_edge_index[0].astype(jnp.int32).reshape(nblk, 1, blk)
    n1r = neg_edge_index[1].astype(jnp.int32).reshape(nblk, 1, blk)
    est3, accs = _loss(posd, negd, e0r, e1r, n0r, n1r)

    reps = reps2[0, :NN]
    est = est3.reshape(e)
    rec_loss = (accs[1] + accs[0]) * NN / (accs[2] + accs[3])
    return reps, rec_loss, est
